# Initial kernel scaffold; baseline (speedup 1.0000x reference)
import jax
import jax.numpy as jnp
from jax.experimental import pallas as pl


def kernel(basic_block, edge_index, h0, c0, gcn_W, gcn_b, W_ih, W_hh, b_ih, b_hh):
    raise NotImplementedError("write your pallas kernel here")


# speedup vs baseline: 14.7506x; 14.7506x over previous
import jax
import jax.numpy as jnp
from jax import lax
from jax.experimental import pallas as pl
from jax.experimental.pallas import tpu as pltpu
from jax.experimental.pallas import tpu_sc as plsc

N = 10000
E = 320000
D = 128
H = 128
G4 = 4 * H

NC = 2
NS = 16
NW = NC * NS
EPW = E // NW
CH = 80
NCH = EPW // CH
WPT = 2
MCH = WPT * NCH
NPAD = 10240
DSL = NPAD // NS
HR = N // NC
APAD = 5120
ASL = APAD // NS
TRASH = 5100
ZR = 64

TB = 1000
NB = N // TB

_f32 = jnp.float32


def _sc_mesh():
    return plsc.VectorSubcoreMesh(core_axis_name="c", subcore_axis_name="s")


def _deg_body(dst_hbm, deg_hbm, idx_v, ones_v, zer_v, deg_sh):
    cid = lax.axis_index("c")
    sid = lax.axis_index("s")
    w = cid * NS + sid

    def fill_ones(k, _):
        ones_v[pl.ds(k * 16, 16)] = jnp.full((16,), 1.0, _f32)
        return 0

    lax.fori_loop(0, CH // 16, fill_ones, 0)

    def fill_zero(k, _):
        zer_v[pl.ds(k * 16, 16)] = jnp.zeros((16,), _f32)
        return 0

    lax.fori_loop(0, DSL // 16, fill_zero, 0)
    pltpu.sync_copy(zer_v, deg_sh.at[pl.ds(sid * DSL, DSL)])
    pltpu.sync_copy(dst_hbm.at[w], idx_v)
    plsc.subcore_barrier()

    def chunk(j, _):
        pltpu.sync_copy(ones_v, deg_sh.at[idx_v.at[j]], add=True)
        return 0

    lax.fori_loop(0, NCH, chunk, 0)
    plsc.subcore_barrier()
    pltpu.sync_copy(deg_sh.at[pl.ds(sid * DSL, DSL)],
                    deg_hbm.at[cid, pl.ds(sid * DSL, DSL)])


def _deg_call(dst3):
    return pl.kernel(
        _deg_body,
        out_type=jax.ShapeDtypeStruct((NC, NPAD), _f32),
        mesh=_sc_mesh(),
        scratch_types=[
            pltpu.VMEM((NCH, CH), jnp.int32),
            pltpu.VMEM((CH,), _f32),
            pltpu.VMEM((DSL,), _f32),
            pltpu.VMEM_SHARED((NPAD,), _f32),
        ],
    )(dst3)


def _msg_body(xw_hbm, src_hbm, dst_hbm, acc_hbm,
              isrc_v, idst_v, rows_v, zrows_v, acc_sh, sem):
    cid = lax.axis_index("c")
    sid = lax.axis_index("s")

    def fill_zero(r, _):
        for k in range(D // 16):
            zrows_v[r, pl.ds(k * 16, 16)] = jnp.zeros((16,), _f32)
        return 0

    lax.fori_loop(0, ZR, fill_zero, 0)

    def zero_out(t, _):
        pltpu.sync_copy(zrows_v, acc_sh.at[pl.ds(sid * ASL + t * ZR, ZR)])
        return 0

    lax.fori_loop(0, ASL // ZR, zero_out, 0)

    for b in range(WPT):
        pltpu.sync_copy(src_hbm.at[WPT * sid + b], isrc_v.at[b])
        pltpu.sync_copy(dst_hbm.at[WPT * sid + b], idst_v.at[b])

    base = cid * HR
    for b in range(WPT):
        def remap(r, _):
            for k in range(CH // 16):
                v = idst_v[b, r, pl.ds(k * 16, 16)] - base
                ok = (v >= 0) & (v < HR)
                idst_v[b, r, pl.ds(k * 16, 16)] = jnp.where(
                    ok, v, jnp.full((16,), TRASH, jnp.int32))
            return 0

        lax.fori_loop(0, NCH, remap, 0)
    plsc.subcore_barrier()

    for b in range(WPT):
        def chunk(j, _):
            pltpu.async_copy(xw_hbm.at[isrc_v.at[b, j]], rows_v, sem).wait()
            pltpu.sync_copy(rows_v, acc_sh.at[idst_v.at[b, j]], add=True)
            return 0

        lax.fori_loop(0, NCH, chunk, 0)
    plsc.subcore_barrier()
    pltpu.sync_copy(acc_sh.at[pl.ds(sid * ASL, ASL)],
                    acc_hbm.at[cid, pl.ds(sid * ASL, ASL)])


def _msg_call(xwp, src3, dst3):
    return pl.kernel(
        _msg_body,
        out_type=jax.ShapeDtypeStruct((NC, APAD, D), _f32),
        mesh=_sc_mesh(),
        scratch_types=[
            pltpu.VMEM((WPT, NCH, CH), jnp.int32),
            pltpu.VMEM((WPT, NCH, CH), jnp.int32),
            pltpu.VMEM((CH, D), _f32),
            pltpu.VMEM((ZR, D), _f32),
            pltpu.VMEM_SHARED((APAD, D), _f32),
            pltpu.SemaphoreType.DMA,
        ],
    )(xwp, src3, dst3)


def _scale_body(bb_ref, w_ref, deg_ref, xwp_ref, dinv_ref):
    deg = deg_ref[0] + deg_ref[1] + 1.0
    dinv = lax.rsqrt(deg)
    xw = jnp.dot(bb_ref[...], w_ref[...], preferred_element_type=_f32)
    xwp_ref[...] = xw * dinv
    dinv_ref[...] = dinv


def _scale_call(bb, gcn_W, deg2):
    return pl.pallas_call(
        _scale_body,
        grid=(NB,),
        in_specs=[
            pl.BlockSpec((TB, D), lambda i: (i, 0)),
            pl.BlockSpec((D, H), lambda i: (0, 0)),
            pl.BlockSpec((NC, TB, 1), lambda i: (0, i, 0)),
        ],
        out_specs=[
            pl.BlockSpec((TB, H), lambda i: (i, 0)),
            pl.BlockSpec((TB, 1), lambda i: (i, 0)),
        ],
        out_shape=[
            jax.ShapeDtypeStruct((N, H), _f32),
            jax.ShapeDtypeStruct((N, 1), _f32),
        ],
    )(bb, gcn_W, deg2)


def _lstm_body(acc_ref, xwp_ref, dinv_ref, b_ref, wihT_ref, whhT_ref,
               bsum_ref, h0_ref, c0_ref, ys_ref, hN_ref, cN_ref, a_scr):
    i = pl.program_id(0)

    @pl.when(i == 0)
    def _():
        hN_ref[...] = h0_ref[...]
        cN_ref[...] = c0_ref[...]

    x = (acc_ref[0] + xwp_ref[...]) * dinv_ref[...] + b_ref[...]
    a_scr[...] = jnp.dot(x, wihT_ref[...], preferred_element_type=_f32) + bsum_ref[...]

    def step(t, hc):
        h, c = hc
        g = a_scr[pl.ds(t, 1), :] + jnp.dot(h, whhT_ref[...],
                                            preferred_element_type=_f32)
        s = jax.nn.sigmoid(g)
        ig = s[:, 0:H]
        fg = s[:, H:2 * H]
        gg = 2.0 * s[:, 2 * H:3 * H] - 1.0
        og = s[:, 3 * H:G4]
        c2 = fg * c + ig * gg
        h2 = og * jnp.tanh(c2)
        ys_ref[pl.ds(t, 1), :] = h2
        return (h2, c2)

    h, c = lax.fori_loop(0, TB, step, (hN_ref[...], cN_ref[...]))
    hN_ref[...] = h
    cN_ref[...] = c


def _lstm_call(acc, xwp, dinv, b, wihT, whhT, bsum, h0, c0):
    nhb = NB // NC
    return pl.pallas_call(
        _lstm_body,
        grid=(NB,),
        in_specs=[
            pl.BlockSpec((1, TB, D), lambda i: (i // nhb, i % nhb, 0)),
            pl.BlockSpec((TB, D), lambda i: (i, 0)),
            pl.BlockSpec((TB, 1), lambda i: (i, 0)),
            pl.BlockSpec((1, D), lambda i: (0, 0)),
            pl.BlockSpec((D, G4), lambda i: (0, 0)),
            pl.BlockSpec((H, G4), lambda i: (0, 0)),
            pl.BlockSpec((1, G4), lambda i: (0, 0)),
            pl.BlockSpec((1, H), lambda i: (0, 0)),
            pl.BlockSpec((1, H), lambda i: (0, 0)),
        ],
        out_specs=[
            pl.BlockSpec((TB, H), lambda i: (i, 0)),
            pl.BlockSpec((1, H), lambda i: (0, 0)),
            pl.BlockSpec((1, H), lambda i: (0, 0)),
        ],
        out_shape=[
            jax.ShapeDtypeStruct((N, H), _f32),
            jax.ShapeDtypeStruct((1, H), _f32),
            jax.ShapeDtypeStruct((1, H), _f32),
        ],
        scratch_shapes=[pltpu.VMEM((TB, G4), _f32)],
    )(acc, xwp, dinv, b, wihT, whhT, bsum, h0, c0)


def kernel(basic_block, edge_index, h0, c0, gcn_W, gcn_b, W_ih, W_hh, b_ih, b_hh):
    src3 = edge_index[0].reshape(NW, NCH, CH)
    dst3 = edge_index[1].reshape(NW, NCH, CH)

    deg2 = _deg_call(dst3)
    deg2 = deg2[:, :N].reshape(NC, N, 1)

    xwp, dinv = _scale_call(basic_block, gcn_W, deg2)

    acc = _msg_call(xwp, src3, dst3)

    gate_scale = jnp.concatenate(
        [jnp.ones((2 * H,), _f32), jnp.full((H,), 2.0, _f32), jnp.ones((H,), _f32)])
    wihT = W_ih.T * gate_scale[None, :]
    whhT = W_hh.T * gate_scale[None, :]
    bsum = ((b_ih + b_hh) * gate_scale).reshape(1, G4)

    ys, hN, cN = _lstm_call(acc, xwp, dinv, gcn_b.reshape(1, D),
                            wihT, whhT, bsum, h0, c0)
    return (ys, hN, cN)


# speedup vs baseline: 16.0956x; 1.0912x over previous
import jax
import jax.numpy as jnp
from jax import lax
from jax.experimental import pallas as pl
from jax.experimental.pallas import tpu as pltpu
from jax.experimental.pallas import tpu_sc as plsc

N = 10000
E = 320000
D = 128
H = 128
G4 = 4 * H

NC = 2
NS = 16
NW = NC * NS
EPW = E // NW
CH = 80
NCH = EPW // CH
WPT = 2
MCH = WPT * NCH
NPAD = 10240
DSL = NPAD // NS
HR = N // NC
APAD = 5120
ASL = APAD // NS
TRASH = 5100
ZR = 64

TB = 1000
NB = N // TB

_f32 = jnp.float32


def _sc_mesh():
    return plsc.VectorSubcoreMesh(core_axis_name="c", subcore_axis_name="s")


def _deg_body(dst_hbm, deg_hbm, idx_v, ones_v, zer_v, deg_sh):
    cid = lax.axis_index("c")
    sid = lax.axis_index("s")
    w = cid * NS + sid

    def fill_ones(k, _):
        ones_v[pl.ds(k * 16, 16)] = jnp.full((16,), 1.0, _f32)
        return 0

    lax.fori_loop(0, CH // 16, fill_ones, 0)

    def fill_zero(k, _):
        zer_v[pl.ds(k * 16, 16)] = jnp.zeros((16,), _f32)
        return 0

    lax.fori_loop(0, DSL // 16, fill_zero, 0)
    pltpu.sync_copy(zer_v, deg_sh.at[pl.ds(sid * DSL, DSL)])
    pltpu.sync_copy(dst_hbm.at[w], idx_v)
    plsc.subcore_barrier()

    def chunk(j, _):
        pltpu.sync_copy(ones_v, deg_sh.at[idx_v.at[j]], add=True)
        return 0

    lax.fori_loop(0, NCH, chunk, 0)
    plsc.subcore_barrier()
    pltpu.sync_copy(deg_sh.at[pl.ds(sid * DSL, DSL)],
                    deg_hbm.at[cid, pl.ds(sid * DSL, DSL)])


def _deg_call(dst3):
    return pl.kernel(
        _deg_body,
        out_type=jax.ShapeDtypeStruct((NC, NPAD), _f32),
        mesh=_sc_mesh(),
        scratch_types=[
            pltpu.VMEM((NCH, CH), jnp.int32),
            pltpu.VMEM((CH,), _f32),
            pltpu.VMEM((DSL,), _f32),
            pltpu.VMEM_SHARED((NPAD,), _f32),
        ],
    )(dst3)


def _msg_body(xw_hbm, src_hbm, dst_hbm, acc_hbm,
              isrc_v, idst_v, rows_v, zrows_v, acc_sh, sem):
    cid = lax.axis_index("c")
    sid = lax.axis_index("s")

    def fill_zero(r, _):
        for k in range(D // 16):
            zrows_v[r, pl.ds(k * 16, 16)] = jnp.zeros((16,), _f32)
        return 0

    lax.fori_loop(0, ZR, fill_zero, 0)

    def zero_out(t, _):
        pltpu.sync_copy(zrows_v, acc_sh.at[pl.ds(sid * ASL + t * ZR, ZR)])
        return 0

    lax.fori_loop(0, ASL // ZR, zero_out, 0)

    for b in range(WPT):
        pltpu.sync_copy(src_hbm.at[WPT * sid + b], isrc_v.at[b])
        pltpu.sync_copy(dst_hbm.at[WPT * sid + b], idst_v.at[b])

    base = cid * HR
    for b in range(WPT):
        def remap(r, _):
            for k in range(CH // 16):
                v = idst_v[b, r, pl.ds(k * 16, 16)] - base
                ok = (v >= 0) & (v < HR)
                idst_v[b, r, pl.ds(k * 16, 16)] = jnp.where(
                    ok, v, jnp.full((16,), TRASH, jnp.int32))
            return 0

        lax.fori_loop(0, NCH, remap, 0)
    plsc.subcore_barrier()

    for b in range(WPT):
        def chunk(j, _):
            pltpu.async_copy(xw_hbm.at[isrc_v.at[b, j]], rows_v, sem).wait()
            pltpu.sync_copy(rows_v, acc_sh.at[idst_v.at[b, j]], add=True)
            return 0

        lax.fori_loop(0, NCH, chunk, 0)
    plsc.subcore_barrier()
    pltpu.sync_copy(acc_sh.at[pl.ds(sid * ASL, ASL)],
                    acc_hbm.at[cid, pl.ds(sid * ASL, ASL)])


def _msg_call(xwp, src3, dst3):
    return pl.kernel(
        _msg_body,
        out_type=jax.ShapeDtypeStruct((NC, APAD, D), _f32),
        mesh=_sc_mesh(),
        scratch_types=[
            pltpu.VMEM((WPT, NCH, CH), jnp.int32),
            pltpu.VMEM((WPT, NCH, CH), jnp.int32),
            pltpu.VMEM((CH, D), _f32),
            pltpu.VMEM((ZR, D), _f32),
            pltpu.VMEM_SHARED((APAD, D), _f32),
            pltpu.SemaphoreType.DMA,
        ],
    )(xwp, src3, dst3)


def _scale_body(bb_ref, w_ref, deg_ref, xwp_ref, dinv_ref):
    deg = deg_ref[0] + deg_ref[1] + 1.0
    dinv = lax.rsqrt(deg)
    xw = jnp.dot(bb_ref[...], w_ref[...], preferred_element_type=_f32)
    xwp_ref[...] = xw * dinv
    dinv_ref[...] = dinv


def _scale_call(bb, gcn_W, deg2):
    return pl.pallas_call(
        _scale_body,
        grid=(NB,),
        in_specs=[
            pl.BlockSpec((TB, D), lambda i: (i, 0)),
            pl.BlockSpec((D, H), lambda i: (0, 0)),
            pl.BlockSpec((NC, TB, 1), lambda i: (0, i, 0)),
        ],
        out_specs=[
            pl.BlockSpec((TB, H), lambda i: (i, 0)),
            pl.BlockSpec((TB, 1), lambda i: (i, 0)),
        ],
        out_shape=[
            jax.ShapeDtypeStruct((N, H), _f32),
            jax.ShapeDtypeStruct((N, 1), _f32),
        ],
    )(bb, gcn_W, deg2)


def _lstm_body(acc_ref, xwp_ref, dinv_ref, b_ref, wihT_ref, whhT_ref,
               bsum_ref, h0_ref, c0_ref, ys_ref, hN_ref, cN_ref, a_scr):
    i = pl.program_id(0)

    @pl.when(i == 0)
    def _():
        hN_ref[...] = h0_ref[...]
        cN_ref[...] = c0_ref[...]

    x = (acc_ref[0] + xwp_ref[...]) * dinv_ref[...] + b_ref[...]
    a_scr[...] = jnp.dot(x, wihT_ref[...], preferred_element_type=_f32) + bsum_ref[...]

    def step8(tt, hc):
        h, c = hc
        base = tt * 8
        for k in range(8):
            g = a_scr[pl.ds(base + k, 1), :] + jnp.dot(
                h, whhT_ref[...], preferred_element_type=_f32)
            s = jax.nn.sigmoid(g)
            ig = s[:, 0:H]
            fg = s[:, H:2 * H]
            gg = 2.0 * s[:, 2 * H:3 * H] - 1.0
            og = s[:, 3 * H:G4]
            c = fg * c + ig * gg
            h = og * jnp.tanh(c)
            ys_ref[pl.ds(base + k, 1), :] = h
        return (h, c)

    h, c = lax.fori_loop(0, TB // 8, step8, (hN_ref[...], cN_ref[...]))
    hN_ref[...] = h
    cN_ref[...] = c


def _lstm_call(acc, xwp, dinv, b, wihT, whhT, bsum, h0, c0):
    nhb = NB // NC
    return pl.pallas_call(
        _lstm_body,
        grid=(NB,),
        in_specs=[
            pl.BlockSpec((1, TB, D), lambda i: (i // nhb, i % nhb, 0)),
            pl.BlockSpec((TB, D), lambda i: (i, 0)),
            pl.BlockSpec((TB, 1), lambda i: (i, 0)),
            pl.BlockSpec((1, D), lambda i: (0, 0)),
            pl.BlockSpec((D, G4), lambda i: (0, 0)),
            pl.BlockSpec((H, G4), lambda i: (0, 0)),
            pl.BlockSpec((1, G4), lambda i: (0, 0)),
            pl.BlockSpec((1, H), lambda i: (0, 0)),
            pl.BlockSpec((1, H), lambda i: (0, 0)),
        ],
        out_specs=[
            pl.BlockSpec((TB, H), lambda i: (i, 0)),
            pl.BlockSpec((1, H), lambda i: (0, 0)),
            pl.BlockSpec((1, H), lambda i: (0, 0)),
        ],
        out_shape=[
            jax.ShapeDtypeStruct((N, H), _f32),
            jax.ShapeDtypeStruct((1, H), _f32),
            jax.ShapeDtypeStruct((1, H), _f32),
        ],
        scratch_shapes=[pltpu.VMEM((TB, G4), _f32)],
    )(acc, xwp, dinv, b, wihT, whhT, bsum, h0, c0)


def kernel(basic_block, edge_index, h0, c0, gcn_W, gcn_b, W_ih, W_hh, b_ih, b_hh):
    src3 = edge_index[0].reshape(NW, NCH, CH)
    dst3 = edge_index[1].reshape(NW, NCH, CH)

    deg2 = _deg_call(dst3)
    deg2 = deg2[:, :N].reshape(NC, N, 1)

    xwp, dinv = _scale_call(basic_block, gcn_W, deg2)

    acc = _msg_call(xwp, src3, dst3)

    gate_scale = jnp.concatenate(
        [jnp.ones((2 * H,), _f32), jnp.full((H,), 2.0, _f32), jnp.ones((H,), _f32)])
    wihT = W_ih.T * gate_scale[None, :]
    whhT = W_hh.T * gate_scale[None, :]
    bsum = ((b_ih + b_hh) * gate_scale).reshape(1, G4)

    ys, hN, cN = _lstm_call(acc, xwp, dinv, gcn_b.reshape(1, D),
                            wihT, whhT, bsum, h0, c0)
    return (ys, hN, cN)


# speedup vs baseline: 16.3918x; 1.0184x over previous
import jax
import jax.numpy as jnp
from jax import lax
from jax.experimental import pallas as pl
from jax.experimental.pallas import tpu as pltpu
from jax.experimental.pallas import tpu_sc as plsc

N = 10000
E = 320000
D = 128
H = 128
G4 = 4 * H

NC = 2
NS = 16
NW = NC * NS
EPW = E // NW
CH = 80
NCH = EPW // CH
WPT = 2
MCH = WPT * NCH
NPAD = 10240
DSL = NPAD // NS
HR = N // NC
APAD = 5120
ASL = APAD // NS
TRASH = 5100
ZR = 64

TB = 1000
NB = N // TB

_f32 = jnp.float32


def _sc_mesh():
    return plsc.VectorSubcoreMesh(core_axis_name="c", subcore_axis_name="s")


def _deg_body(dst_hbm, deg_hbm, idx_v, ones_v, zer_v, deg_sh):
    cid = lax.axis_index("c")
    sid = lax.axis_index("s")
    w = cid * NS + sid

    def fill_ones(k, _):
        ones_v[pl.ds(k * 16, 16)] = jnp.full((16,), 1.0, _f32)
        return 0

    lax.fori_loop(0, CH // 16, fill_ones, 0)

    def fill_zero(k, _):
        zer_v[pl.ds(k * 16, 16)] = jnp.zeros((16,), _f32)
        return 0

    lax.fori_loop(0, DSL // 16, fill_zero, 0)
    pltpu.sync_copy(zer_v, deg_sh.at[pl.ds(sid * DSL, DSL)])
    pltpu.sync_copy(dst_hbm.at[w], idx_v)
    plsc.subcore_barrier()

    def chunk(j, _):
        pltpu.sync_copy(ones_v, deg_sh.at[idx_v.at[j]], add=True)
        return 0

    lax.fori_loop(0, NCH, chunk, 0)
    plsc.subcore_barrier()
    pltpu.sync_copy(deg_sh.at[pl.ds(sid * DSL, DSL)],
                    deg_hbm.at[cid, pl.ds(sid * DSL, DSL)])


def _deg_call(dst3):
    return pl.kernel(
        _deg_body,
        out_type=jax.ShapeDtypeStruct((NC, NPAD), _f32),
        mesh=_sc_mesh(),
        scratch_types=[
            pltpu.VMEM((NCH, CH), jnp.int32),
            pltpu.VMEM((CH,), _f32),
            pltpu.VMEM((DSL,), _f32),
            pltpu.VMEM_SHARED((NPAD,), _f32),
        ],
    )(dst3)


def _msg_body(xw_hbm, src_hbm, dst_hbm, acc_hbm,
              isrc_v, idst_v, rows_v, zrows_v, acc_sh, sem):
    cid = lax.axis_index("c")
    sid = lax.axis_index("s")

    def fill_zero(r, _):
        for k in range(D // 16):
            zrows_v[r, pl.ds(k * 16, 16)] = jnp.zeros((16,), _f32)
        return 0

    lax.fori_loop(0, ZR, fill_zero, 0)

    def zero_out(t, _):
        pltpu.sync_copy(zrows_v, acc_sh.at[pl.ds(sid * ASL + t * ZR, ZR)])
        return 0

    lax.fori_loop(0, ASL // ZR, zero_out, 0)

    for b in range(WPT):
        pltpu.sync_copy(src_hbm.at[WPT * sid + b], isrc_v.at[b])
        pltpu.sync_copy(dst_hbm.at[WPT * sid + b], idst_v.at[b])

    base = cid * HR
    for b in range(WPT):
        def remap(r, _):
            for k in range(CH // 16):
                v = idst_v[b, r, pl.ds(k * 16, 16)] - base
                ok = (v >= 0) & (v < HR)
                idst_v[b, r, pl.ds(k * 16, 16)] = jnp.where(
                    ok, v, jnp.full((16,), TRASH, jnp.int32))
            return 0

        lax.fori_loop(0, NCH, remap, 0)
    plsc.subcore_barrier()

    for b in range(WPT):
        def chunk(j, _):
            pltpu.async_copy(xw_hbm.at[isrc_v.at[b, j]], rows_v, sem).wait()
            pltpu.sync_copy(rows_v, acc_sh.at[idst_v.at[b, j]], add=True)
            return 0

        lax.fori_loop(0, NCH, chunk, 0)
    plsc.subcore_barrier()
    pltpu.sync_copy(acc_sh.at[pl.ds(sid * ASL, ASL)],
                    acc_hbm.at[cid, pl.ds(sid * ASL, ASL)])


def _msg_call(xwp, src3, dst3):
    return pl.kernel(
        _msg_body,
        out_type=jax.ShapeDtypeStruct((NC, APAD, D), _f32),
        mesh=_sc_mesh(),
        scratch_types=[
            pltpu.VMEM((WPT, NCH, CH), jnp.int32),
            pltpu.VMEM((WPT, NCH, CH), jnp.int32),
            pltpu.VMEM((CH, D), _f32),
            pltpu.VMEM((ZR, D), _f32),
            pltpu.VMEM_SHARED((APAD, D), _f32),
            pltpu.SemaphoreType.DMA,
        ],
    )(xwp, src3, dst3)


def _scale_body(bb_ref, w_ref, deg_ref, xwp_ref, dinv_ref):
    deg = deg_ref[0] + deg_ref[1] + 1.0
    dinv = lax.rsqrt(deg)
    xw = jnp.dot(bb_ref[...], w_ref[...], preferred_element_type=_f32)
    xwp_ref[...] = xw * dinv
    dinv_ref[...] = dinv


def _scale_call(bb, gcn_W, deg2):
    return pl.pallas_call(
        _scale_body,
        grid=(NB,),
        in_specs=[
            pl.BlockSpec((TB, D), lambda i: (i, 0)),
            pl.BlockSpec((D, H), lambda i: (0, 0)),
            pl.BlockSpec((NC, TB, 1), lambda i: (0, i, 0)),
        ],
        out_specs=[
            pl.BlockSpec((TB, H), lambda i: (i, 0)),
            pl.BlockSpec((TB, 1), lambda i: (i, 0)),
        ],
        out_shape=[
            jax.ShapeDtypeStruct((N, H), _f32),
            jax.ShapeDtypeStruct((N, 1), _f32),
        ],
    )(bb, gcn_W, deg2)


def _lstm_body(acc_ref, xwp_ref, dinv_ref, b_ref, wihT_ref, whhT_ref,
               bsum_ref, h0_ref, c0_ref, ys_ref, hN_ref, cN_ref, a_scr):
    i = pl.program_id(0)

    @pl.when(i == 0)
    def _():
        hN_ref[...] = h0_ref[...]
        cN_ref[...] = c0_ref[...]

    x = (acc_ref[0] + xwp_ref[...]) * dinv_ref[...] + b_ref[...]
    a_scr[...] = jnp.dot(x, wihT_ref[...], preferred_element_type=_f32) + bsum_ref[...]

    def step8(tt, hc):
        h, c = hc
        base = tt * 8
        a8 = a_scr[pl.ds(base, 8), :]
        hs = []
        for k in range(8):
            g = a8[k:k + 1, :] + jnp.dot(
                h, whhT_ref[...], preferred_element_type=_f32)
            s = jax.nn.sigmoid(g)
            ig = s[:, 0:H]
            fg = s[:, H:2 * H]
            gg = 2.0 * s[:, 2 * H:3 * H] - 1.0
            og = s[:, 3 * H:G4]
            c = fg * c + ig * gg
            h = og * jnp.tanh(c)
            hs.append(h)
        ys_ref[pl.ds(base, 8), :] = jnp.concatenate(hs, axis=0)
        return (h, c)

    h, c = lax.fori_loop(0, TB // 8, step8, (hN_ref[...], cN_ref[...]))
    hN_ref[...] = h
    cN_ref[...] = c


def _lstm_call(acc, xwp, dinv, b, wihT, whhT, bsum, h0, c0):
    nhb = NB // NC
    return pl.pallas_call(
        _lstm_body,
        grid=(NB,),
        in_specs=[
            pl.BlockSpec((1, TB, D), lambda i: (i // nhb, i % nhb, 0)),
            pl.BlockSpec((TB, D), lambda i: (i, 0)),
            pl.BlockSpec((TB, 1), lambda i: (i, 0)),
            pl.BlockSpec((1, D), lambda i: (0, 0)),
            pl.BlockSpec((D, G4), lambda i: (0, 0)),
            pl.BlockSpec((H, G4), lambda i: (0, 0)),
            pl.BlockSpec((1, G4), lambda i: (0, 0)),
            pl.BlockSpec((1, H), lambda i: (0, 0)),
            pl.BlockSpec((1, H), lambda i: (0, 0)),
        ],
        out_specs=[
            pl.BlockSpec((TB, H), lambda i: (i, 0)),
            pl.BlockSpec((1, H), lambda i: (0, 0)),
            pl.BlockSpec((1, H), lambda i: (0, 0)),
        ],
        out_shape=[
            jax.ShapeDtypeStruct((N, H), _f32),
            jax.ShapeDtypeStruct((1, H), _f32),
            jax.ShapeDtypeStruct((1, H), _f32),
        ],
        scratch_shapes=[pltpu.VMEM((TB, G4), _f32)],
    )(acc, xwp, dinv, b, wihT, whhT, bsum, h0, c0)


def kernel(basic_block, edge_index, h0, c0, gcn_W, gcn_b, W_ih, W_hh, b_ih, b_hh):
    src3 = edge_index[0].reshape(NW, NCH, CH)
    dst3 = edge_index[1].reshape(NW, NCH, CH)

    deg2 = _deg_call(dst3)
    deg2 = deg2[:, :N].reshape(NC, N, 1)

    xwp, dinv = _scale_call(basic_block, gcn_W, deg2)

    acc = _msg_call(xwp, src3, dst3)

    gate_scale = jnp.concatenate(
        [jnp.ones((2 * H,), _f32), jnp.full((H,), 2.0, _f32), jnp.ones((H,), _f32)])
    wihT = W_ih.T * gate_scale[None, :]
    whhT = W_hh.T * gate_scale[None, :]
    bsum = ((b_ih + b_hh) * gate_scale).reshape(1, G4)

    ys, hN, cN = _lstm_call(acc, xwp, dinv, gcn_b.reshape(1, D),
                            wihT, whhT, bsum, h0, c0)
    return (ys, hN, cN)
